# Initial kernel scaffold; baseline (speedup 1.0000x reference)
import jax
import jax.numpy as jnp
from jax.experimental import pallas as pl


def kernel(x, mask_token):
    raise NotImplementedError("write your pallas kernel here")


# speedup vs baseline: 15.5052x; 15.5052x over previous
import jax
import jax.numpy as jnp
from jax.experimental import pallas as pl
from jax.experimental.pallas import tpu as pltpu

_NT = 576
_NV = 144


def _body(jlt_ref, nr_ref, nc_ref, x_ref, mt_ref, out_ref, mask_ref):
    nr = nr_ref[0]
    nc = nc_ref[0]
    lt = (nr < nc).astype(jnp.float32)
    eq = (nr == nc).astype(jnp.float32)
    cnt = lt + eq * jlt_ref[...]
    rank = jnp.sum(cnt, axis=1, keepdims=True)
    masked = rank >= float(_NV)
    mask_ref[0] = masked.astype(jnp.float32)
    out_ref[0] = jnp.where(masked, mt_ref[...], x_ref[0])


def kernel(x, mask_token):
    b, d = x.shape[0], x.shape[-1]
    noise = jax.random.uniform(jax.random.key(42), (b, 1, _NT), dtype=jnp.float32)
    noise_c = noise.reshape(b, _NT, 1)
    jlt = (jax.lax.broadcasted_iota(jnp.int32, (_NT, _NT), 1)
           < jax.lax.broadcasted_iota(jnp.int32, (_NT, _NT), 0)).astype(jnp.float32)
    x_flat = x.reshape(b, _NT, d)

    out, mask3 = pl.pallas_call(
        _body,
        grid=(b,),
        in_specs=[
            pl.BlockSpec((_NT, _NT), lambda i: (0, 0)),
            pl.BlockSpec((1, 1, _NT), lambda i: (i, 0, 0)),
            pl.BlockSpec((1, _NT, 1), lambda i: (i, 0, 0)),
            pl.BlockSpec((1, _NT, d), lambda i: (i, 0, 0)),
            pl.BlockSpec((1, d), lambda i: (0, 0)),
        ],
        out_specs=[
            pl.BlockSpec((1, _NT, d), lambda i: (i, 0, 0)),
            pl.BlockSpec((1, _NT, 1), lambda i: (i, 0, 0)),
        ],
        out_shape=[
            jax.ShapeDtypeStruct((b, _NT, d), jnp.float32),
            jax.ShapeDtypeStruct((b, _NT, 1), jnp.float32),
        ],
        compiler_params=pltpu.CompilerParams(
            dimension_semantics=("arbitrary",),
        ),
    )(jlt, noise, noise_c, x_flat, mask_token.reshape(1, d))

    return out.reshape(x.shape), mask3.reshape(b, _NT)


# speedup vs baseline: 21.1132x; 1.3617x over previous
import functools
import jax
import jax.numpy as jnp
from jax import lax
from jax.experimental import pallas as pl
from jax.experimental.pallas import tpu as pltpu
from jax.experimental.pallas import tpu_sc as plsc

_NT = 576
_NV = 144
_NM = _NT - _NV
_B = 64
_ROWS = _B * _NT
_CH = 48
_NVC = _NV // _CH
_NMC = _NM // _CH
_NVEC = _NT // 16


def _sc_body(u_hbm, mt_hbm, x_hbm, out_hbm, masks_hbm,
             u_v, masks_v, vis_flat, msk_flat,
             vi0, vi1, vi2, mi0, mi1, mi2, mi3, mi4, mi5, mi6, mi7, mi8,
             buf0, buf1, mrow,
             semi, semg, semv, semm, semr):
    vis_refs = (vi0, vi1, vi2)
    msk_refs = (mi0, mi1, mi2, mi3, mi4, mi5, mi6, mi7, mi8)
    bufs = (buf0, buf1)
    wid = lax.axis_index("s") * 2 + lax.axis_index("c")

    hs = [pltpu.async_copy(mt_hbm, mrow.at[r], semi) for r in range(_CH)]
    for h in hs:
        h.wait()

    lane = lax.iota(jnp.int32, 16)

    def do_batch(b):
        pltpu.sync_copy(u_hbm.at[pl.ds(b * _NT, _NT)], u_v)

        def cnt_le(mid):
            acc = jnp.zeros((16,), jnp.int32)
            midv = jnp.full((16,), mid, jnp.int32)
            for i in range(_NVEC):
                uv = u_v[pl.ds(i * 16, 16)]
                acc = acc + (uv <= midv).astype(jnp.int32)
            return jnp.sum(acc)

        @pl.loop(0, 30, init_carry=(jnp.int32(0), jnp.int32(0x3F800000)))
        def bisect(_, c):
            lo, hi = c
            mid = lax.div(lo + hi, 2)
            big = cnt_le(mid) >= _NV
            return (jnp.where(big, lo, mid + 1), jnp.where(big, mid, hi))

        lo, _ = bisect
        vstar = jnp.full((16,), lo, jnp.int32)

        acc = jnp.zeros((16,), jnp.int32)
        for i in range(_NVEC):
            uv = u_v[pl.ds(i * 16, 16)]
            acc = acc + (uv < vstar).astype(jnp.int32)
        n_strict = jnp.sum(acc)
        rem = _NV - n_strict

        ecarry = jnp.int32(0)
        vcarry = jnp.int32(0)
        mcarry = jnp.int32(0)
        remv = jnp.full((16,), rem, jnp.int32)
        for i in range(_NVEC):
            uv = u_v[pl.ds(i * 16, 16)]
            meq = uv == vstar
            e = meq.astype(jnp.int32)
            ce = lax.cumsum(e)
            tie_idx = jnp.full((16,), ecarry, jnp.int32) + ce - e
            vis = (uv < vstar) | (meq & (tie_idx < remv))
            ecarry = ecarry + jnp.sum(e)
            v = vis.astype(jnp.int32)
            masks_v[pl.ds(i * 16, 16)] = 1.0 - vis.astype(jnp.float32)
            ids = jnp.full((16,), b * _NT + i * 16, jnp.int32) + lane
            cv = lax.cumsum(v)
            pos_v = jnp.full((16,), vcarry, jnp.int32) + cv - v
            plsc.store_scatter(vis_flat, [pos_v], ids, mask=vis)
            vcarry = vcarry + jnp.sum(v)
            m = 1 - v
            cm = lax.cumsum(m)
            pos_m = jnp.full((16,), mcarry, jnp.int32) + cm - m
            plsc.store_scatter(msk_flat, [pos_m], ids, mask=~vis)
            mcarry = mcarry + jnp.sum(m)

        hr = pltpu.async_copy(masks_v, masks_hbm.at[pl.ds(b * _NT, _NT)], semr)

        for c in range(_NVC):
            for k in range(_CH // 16):
                vis_refs[c][pl.ds(k * 16, 16)] = vis_flat[pl.ds(c * _CH + k * 16, 16)]
        for c in range(_NMC):
            for k in range(_CH // 16):
                msk_refs[c][pl.ds(k * 16, 16)] = msk_flat[pl.ds(c * _CH + k * 16, 16)]

        hm = [pltpu.async_copy(mrow, out_hbm.at[msk_refs[c]], semm)
              for c in range(_NMC)]

        hg = {0: pltpu.async_copy(x_hbm.at[vis_refs[0]], bufs[0], semg)}
        hv = {}
        for c in range(_NVC):
            if c + 1 < _NVC:
                if c >= 1:
                    hv[c - 1].wait()
                hg[c + 1] = pltpu.async_copy(
                    x_hbm.at[vis_refs[c + 1]], bufs[(c + 1) % 2], semg)
            hg[c].wait()
            hv[c] = pltpu.async_copy(bufs[c % 2], out_hbm.at[vis_refs[c]], semv)

        for h in hm:
            h.wait()
        for c in range(_NVC):
            if c >= _NVC - 2:
                hv[c].wait()
        hr.wait()

    do_batch(wid * 2)
    do_batch(wid * 2 + 1)


def kernel(x, mask_token):
    b, d = x.shape[0], x.shape[-1]
    noise = jax.random.uniform(jax.random.key(42), (b, 1, _NT), dtype=jnp.float32)
    u = lax.bitcast_convert_type(noise.reshape(b * _NT), jnp.int32)
    x_rows = x.reshape(_ROWS, d)

    mesh = plsc.VectorSubcoreMesh(core_axis_name="c", subcore_axis_name="s")
    fn = functools.partial(
        pl.kernel,
        mesh=mesh,
        compiler_params=pltpu.CompilerParams(needs_layout_passes=False),
        out_type=[
            jax.ShapeDtypeStruct((_ROWS, d), jnp.float32),
            jax.ShapeDtypeStruct((_ROWS,), jnp.float32),
        ],
        scratch_types=[
            pltpu.VMEM((_NT,), jnp.int32),
            pltpu.VMEM((_NT,), jnp.float32),
            pltpu.VMEM((_NV + 16,), jnp.int32),
            pltpu.VMEM((_NM + 16,), jnp.int32),
        ] + [pltpu.VMEM((_CH,), jnp.int32) for _ in range(_NVC + _NMC)]
        + [
            pltpu.VMEM((_CH, d), jnp.float32),
            pltpu.VMEM((_CH, d), jnp.float32),
            pltpu.VMEM((_CH, d), jnp.float32),
            pltpu.SemaphoreType.DMA,
            pltpu.SemaphoreType.DMA,
            pltpu.SemaphoreType.DMA,
            pltpu.SemaphoreType.DMA,
            pltpu.SemaphoreType.DMA,
        ],
    )(_sc_body)
    out, masks = fn(u, mask_token, x_rows)
    return out.reshape(x.shape), masks.reshape(b, _NT)
